# Initial kernel scaffold; baseline (speedup 1.0000x reference)
import jax
import jax.numpy as jnp
from jax.experimental import pallas as pl


def kernel(h, W_exp, b_exp, W_gate):
    raise NotImplementedError("write your pallas kernel here")


# speedup vs baseline: 1.7190x; 1.7190x over previous
import functools

import jax
import jax.numpy as jnp
from jax.experimental import pallas as pl
from jax.experimental.pallas import tpu as pltpu

B = 8192
HID = 2048
P = 1024
K = 8
TOPK = 2

BM = 512


def _moe_body(h_ref, wg_ref, w_ref, b_ref, out_ref):
    h32 = h_ref[...]
    gate = jax.lax.dot(h32, wg_ref[...], preferred_element_type=jnp.float32)

    iota = jax.lax.broadcasted_iota(jnp.int32, gate.shape, 1)
    v1 = jnp.max(gate, axis=1, keepdims=True)
    i1 = jnp.min(jnp.where(gate == v1, iota, K), axis=1, keepdims=True)
    masked = jnp.where(iota == i1, -jnp.inf, gate)
    v2 = jnp.max(masked, axis=1, keepdims=True)
    i2 = jnp.min(jnp.where(masked == v2, iota, K), axis=1, keepdims=True)
    t = jnp.exp(v2 - v1)
    w1 = 1.0 / (1.0 + t)
    w2 = t / (1.0 + t)

    hb = h32.astype(jnp.bfloat16)
    acc = jnp.zeros((h32.shape[0], P), jnp.float32)
    for k in range(K):
        wk = jnp.where(i1 == k, w1, 0.0) + jnp.where(i2 == k, w2, 0.0)
        yk = jax.lax.dot(
            hb, w_ref[:, k * P:(k + 1) * P], preferred_element_type=jnp.float32
        )
        acc = acc + wk * (yk + b_ref[k, :][None, :])
    out_ref[...] = acc


@jax.jit
def kernel(h, W_exp, b_exp, W_gate):
    Wb = W_exp.astype(jnp.bfloat16)
    b2 = b_exp.reshape(K, P)
    grid = (B // BM,)
    return pl.pallas_call(
        _moe_body,
        grid=grid,
        in_specs=[
            pl.BlockSpec((BM, HID), lambda i: (i, 0)),
            pl.BlockSpec((HID, K), lambda i: (0, 0)),
            pl.BlockSpec((HID, K * P), lambda i: (0, 0)),
            pl.BlockSpec((K, P), lambda i: (0, 0)),
        ],
        out_specs=pl.BlockSpec((BM, P), lambda i: (i, 0)),
        out_shape=jax.ShapeDtypeStruct((B, P), jnp.float32),
        compiler_params=pltpu.CompilerParams(
            vmem_limit_bytes=60 * 1024 * 1024,
        ),
    )(h, W_gate, Wb, b2)
